# baseline (speedup 1.0000x reference)
import jax
import jax.numpy as jnp
from jax.experimental import pallas as pl

_T = 0.07
_BN = 2048


def _mm_kernel(x_ref, m_ref, o_ref):
    xs = x_ref[...] * (1.0 / _T)
    o_ref[...] = jax.lax.dot_general(
        xs, m_ref[...],
        dimension_numbers=(((1,), (1,)), ((), ())),
        preferred_element_type=jnp.float32)


def kernel(x, y, memory):
    M, K = x.shape
    N = memory.shape[0]
    return pl.pallas_call(
        _mm_kernel,
        grid=(pl.cdiv(N, _BN),),
        in_specs=[
            pl.BlockSpec((M, K), lambda j: (0, 0)),
            pl.BlockSpec((_BN, K), lambda j: (j, 0)),
        ],
        out_specs=pl.BlockSpec((M, _BN), lambda j: (0, j)),
        out_shape=jax.ShapeDtypeStruct((M, N), jnp.float32),
    )(x, memory)


# speedup vs baseline: 1.0955x; 1.0955x over previous
import jax
import jax.numpy as jnp
from jax.experimental import pallas as pl

_T = 0.07
_BM = 32


def _mm_kernel(x_ref, mt_ref, o_ref):
    xs = x_ref[...] * (1.0 / _T)
    o_ref[...] = jax.lax.dot_general(
        xs, mt_ref[...],
        dimension_numbers=(((1,), (0,)), ((), ())),
        preferred_element_type=jnp.float32)


def kernel(x, y, memory):
    M, K = x.shape
    N = memory.shape[0]
    mt = memory.T
    return pl.pallas_call(
        _mm_kernel,
        grid=(M // _BM,),
        in_specs=[
            pl.BlockSpec((_BM, K), lambda i: (i, 0)),
            pl.BlockSpec((K, N), lambda i: (0, 0)),
        ],
        out_specs=pl.BlockSpec((_BM, N), lambda i: (i, 0)),
        out_shape=jax.ShapeDtypeStruct((M, N), jnp.float32),
    )(x, mt)
